# baseline (device time: 14250 ns/iter reference)
import jax
import jax.numpy as jnp
from jax import lax
from jax.experimental import pallas as pl
from jax.experimental.pallas import tpu as pltpu

N_Z = 4
T = 256
V_LOCAL = 4096


def kernel(x, W, labels):
    labels2 = labels.reshape(T, 1)

    def body(x_ref, w_ref, lbl_ref, out_ref, comm_ref, send_sems, recv_sems):
        my_x = lax.axis_index("x")
        my_y = lax.axis_index("y")
        my_z = lax.axis_index("z")

        xv = x_ref[...].astype(jnp.bfloat16)
        wv = w_ref[...].astype(jnp.bfloat16)
        logits = jnp.dot(xv, wv, preferred_element_type=jnp.float32)

        m = jnp.max(logits, axis=1)
        s = jnp.sum(jnp.exp(logits - m[:, None]), axis=1)
        col = lax.broadcasted_iota(jnp.int32, (T, V_LOCAL), 1)
        idx = lbl_ref[...] - my_z * V_LOCAL
        t = jnp.sum(jnp.where(col == idx, logits, 0.0), axis=1)

        comm_ref[my_z, 0, :] = m
        comm_ref[my_z, 1, :] = s
        comm_ref[my_z, 2, :] = t

        barrier = pltpu.get_barrier_semaphore()
        for k in range(1, N_Z):
            peer = lax.rem(my_z + k, N_Z)
            pl.semaphore_signal(
                barrier, inc=1,
                device_id=(my_x, my_y, peer),
                device_id_type=pl.DeviceIdType.MESH,
            )
        pl.semaphore_wait(barrier, N_Z - 1)

        sends = []
        for k in range(1, N_Z):
            peer = lax.rem(my_z + k, N_Z)
            rdma = pltpu.make_async_remote_copy(
                src_ref=comm_ref.at[my_z],
                dst_ref=comm_ref.at[my_z],
                send_sem=send_sems.at[k - 1],
                recv_sem=recv_sems.at[my_z],
                device_id=(my_x, my_y, peer),
                device_id_type=pl.DeviceIdType.MESH,
            )
            rdma.start()
            sends.append(rdma)

        for k in range(1, N_Z):
            origin = lax.rem(my_z - k + N_Z, N_Z)
            recv = pltpu.make_async_remote_copy(
                src_ref=comm_ref.at[origin],
                dst_ref=comm_ref.at[origin],
                send_sem=send_sems.at[0],
                recv_sem=recv_sems.at[origin],
                device_id=(my_x, my_y, origin),
                device_id_type=pl.DeviceIdType.MESH,
            )
            recv.wait_recv()

        ms = [comm_ref[k, 0, :] for k in range(N_Z)]
        m_g = ms[0]
        for k in range(1, N_Z):
            m_g = jnp.maximum(m_g, ms[k])
        s_g = jnp.zeros((T,), jnp.float32)
        t_g = jnp.zeros((T,), jnp.float32)
        for k in range(N_Z):
            s_g = s_g + comm_ref[k, 1, :] * jnp.exp(ms[k] - m_g)
            t_g = t_g + comm_ref[k, 2, :]
        out_ref[...] = m_g + jnp.log(s_g) - t_g

        for rdma in sends:
            rdma.wait_send()

    return pl.pallas_call(
        body,
        out_shape=jax.ShapeDtypeStruct((T,), jnp.float32),
        in_specs=[
            pl.BlockSpec(memory_space=pltpu.VMEM),
            pl.BlockSpec(memory_space=pltpu.VMEM),
            pl.BlockSpec(memory_space=pltpu.VMEM),
        ],
        out_specs=pl.BlockSpec(memory_space=pltpu.VMEM),
        scratch_shapes=[
            pltpu.VMEM((N_Z, 4, T), jnp.float32),
            pltpu.SemaphoreType.DMA((N_Z - 1,)),
            pltpu.SemaphoreType.DMA((N_Z,)),
        ],
        compiler_params=pltpu.CompilerParams(collective_id=0),
    )(x, W, labels2)
